# baseline (speedup 1.0000x reference)
import functools

import jax
import jax.numpy as jnp
from jax import lax
from jax.experimental import pallas as pl
from jax.experimental.pallas import tpu as pltpu
from jax.experimental.pallas import tpu_sc as plsc

T_ = 2048
H_ = 1024
E_ = 8
I_ = 1408
K_ = 2

BM = 256
PPAD = T_ * K_ + E_ * BM
NTILES = PPAD // BM

NC = 2
NS = 16
NW = NC * NS
LANES = 16

_EPAD = 128


def _router(probs_pad):
    BT = 256

    def body(p_ref, i0_ref, i1_ref, w0_ref, w1_ref):
        p = p_ref[...]
        cols = lax.broadcasted_iota(jnp.int32, (BT, _EPAD), 1)
        m1 = jnp.max(p, axis=1, keepdims=True)
        i1 = jnp.min(jnp.where(p == m1, cols, _EPAD), axis=1, keepdims=True)
        p2 = jnp.where(cols == i1, jnp.float32(-2.0), p)
        m2 = jnp.max(p2, axis=1, keepdims=True)
        i2 = jnp.min(jnp.where(p2 == m2, cols, _EPAD), axis=1, keepdims=True)
        s = m1 + m2
        ones = jnp.ones((BT, _EPAD), jnp.float32)
        i0_ref[...] = i1.astype(jnp.float32)
        i1_ref[...] = i2.astype(jnp.float32)
        w0_ref[...] = (m1 / s) * ones
        w1_ref[...] = (m2 / s) * ones

    return pl.pallas_call(
        body,
        grid=(T_ // BT,),
        in_specs=[pl.BlockSpec((BT, _EPAD), lambda i: (i, 0))],
        out_specs=[pl.BlockSpec((BT, 1), lambda i: (i, 0))] * 2
        + [pl.BlockSpec((BT, _EPAD), lambda i: (i, 0))] * 2,
        out_shape=[jax.ShapeDtypeStruct((T_, 1), jnp.float32)] * 2
        + [jax.ShapeDtypeStruct((T_, _EPAD), jnp.float32)] * 2,
    )(probs_pad)


def _allsum(v, buf):
    t = v
    for s in (1, 2, 4, 8):
        buf[pl.ds(0, LANES)] = t
        buf[pl.ds(LANES, LANES)] = t
        t = t + buf[pl.ds(s, LANES)]
    return t


def _iprefix(v, buf):
    buf[pl.ds(0, LANES)] = jnp.zeros_like(v)
    p = v
    for s in (1, 2, 4, 8):
        buf[pl.ds(LANES, LANES)] = p
        p = p + buf[pl.ds(LANES - s, LANES)]
    return p


def _lane_bcast(v, e, iota, buf):
    sel = jnp.where(iota == e, v, jnp.zeros_like(v))
    return _allsum(sel, buf)


def _sc_dispatch(x, e0f, e1f, w0c, w1c):
    tpw = T_ // NW
    nch = tpw // LANES
    mesh = plsc.VectorSubcoreMesh(core_axis_name="c", subcore_axis_name="s",
                                  num_cores=NC, num_subcores=NS)
    fE = [jnp.float32(e) for e in range(E_)]

    @functools.partial(
        pl.kernel,
        out_type=[
            jax.ShapeDtypeStruct((PPAD, H_), jnp.float32),
            jax.ShapeDtypeStruct((PPAD, _EPAD), jnp.float32),
            jax.ShapeDtypeStruct((T_,), jnp.int32),
            jax.ShapeDtypeStruct((T_,), jnp.int32),
            jax.ShapeDtypeStruct((32,), jnp.int32),
        ],
        mesh=mesh,
        scratch_types=[
            pltpu.VMEM((tpw, H_), jnp.float32),
            pltpu.VMEM((T_,), jnp.float32),
            pltpu.VMEM((T_,), jnp.float32),
            pltpu.VMEM((tpw, _EPAD), jnp.float32),
            pltpu.VMEM((tpw, _EPAD), jnp.float32),
            pltpu.VMEM((tpw,), jnp.int32),
            pltpu.VMEM((tpw,), jnp.int32),
            pltpu.VMEM((32,), jnp.int32),
            pltpu.VMEM((2 * LANES,), jnp.float32),
            pltpu.VMEM((2 * LANES,), jnp.int32),
            pltpu.SemaphoreType.DMA,
            pltpu.SemaphoreType.DMA,
        ],
    )
    def k(x_hbm, e0_hbm, e1_hbm, w0_hbm, w1_hbm,
          xp_hbm, wp_hbm, pos0_hbm, pos1_hbm, te_hbm,
          xr_v, ge0_v, ge1_v, w0_v, w1_v, p0_v, p1_v, tile_v,
          buf, bufi, s0, s1):
        wid = lax.axis_index("s") * NC + lax.axis_index("c")
        base = wid * tpw

        cp = pltpu.async_copy(x_hbm.at[pl.ds(base, tpw)], xr_v, s1)
        pltpu.sync_copy(e0_hbm, ge0_v)
        pltpu.sync_copy(e1_hbm, ge1_v)
        pltpu.sync_copy(w0_hbm.at[pl.ds(base, tpw)], w0_v)
        pltpu.sync_copy(w1_hbm.at[pl.ds(base, tpw)], w1_v)

        iota = jnp.arange(LANES, dtype=jnp.int32)
        zero = jnp.zeros((LANES,), jnp.float32)
        zi = jnp.zeros((LANES,), jnp.int32)

        def scan_body(i, carry):
            v0 = ge0_v[pl.ds(i * LANES, LANES)]
            v1 = ge1_v[pl.ds(i * LANES, LANES)]
            pre = jnp.where((zi + i * LANES) < base, 1.0, 0.0)
            new = []
            for e in range(E_):
                c0, p0, c1, p1 = carry[4 * e:4 * e + 4]
                m0 = jnp.where(v0 == fE[e], 1.0, 0.0)
                m1 = jnp.where(v1 == fE[e], 1.0, 0.0)
                new.extend([c0 + m0, p0 + m0 * pre, c1 + m1, p1 + m1 * pre])
            return tuple(new)

        accs = lax.fori_loop(0, T_ // LANES, scan_body, tuple([zero] * (4 * E_)))

        cnt_vec = zi
        c0b, p0b, p1b = [], [], []
        for e in range(E_):
            c0 = _allsum(accs[4 * e], buf).astype(jnp.int32)
            p0 = _allsum(accs[4 * e + 1], buf).astype(jnp.int32)
            c1 = _allsum(accs[4 * e + 2], buf).astype(jnp.int32)
            p1 = _allsum(accs[4 * e + 3], buf).astype(jnp.int32)
            c0b.append(c0)
            p0b.append(p0)
            p1b.append(p1)
            cnt_vec = jnp.where(iota == e, c0 + c1, cnt_vec)

        cap_vec = (cnt_vec + (BM - 1)) & (-BM)
        bufi[pl.ds(0, LANES)] = zi
        ends_vec = _iprefix(cap_vec, bufi)
        offs_vec = ends_vec - cap_vec
        st0b, st1b = [], []
        for e in range(E_):
            ob = _lane_bcast(offs_vec, e, iota, bufi)
            st0b.append(ob + p0b[e])
            st1b.append(ob + c0b[e] + p1b[e])

        @pl.when(wid == 0)
        def _():
            for c in range(32 // LANES):
                tstart = (iota + c * LANES) * BM
                te_vec = zi
                for e in range(E_ - 1):
                    eb = _lane_bcast(ends_vec, e, iota, bufi)
                    te_vec = te_vec + jnp.where(tstart >= eb, 1, 0)
                tile_v[pl.ds(c * LANES, LANES)] = te_vec
            pltpu.sync_copy(tile_v, te_hbm)

        carry0 = [zero] * E_
        carry1 = [zero] * E_
        for c in range(nch):
            v0 = ge0_v[pl.ds(base + c * LANES, LANES)]
            v1 = ge1_v[pl.ds(base + c * LANES, LANES)]
            pos0c = zi
            pos1c = zi
            for e in range(E_):
                m0 = jnp.where(v0 == fE[e], 1.0, 0.0)
                cs0 = _iprefix(m0, buf)
                r0 = cs0 - m0 + carry0[e]
                pos0c = pos0c + ((st0b[e] + r0.astype(jnp.int32))
                                 * m0.astype(jnp.int32))
                carry0[e] = carry0[e] + _lane_bcast(cs0, LANES - 1, iota, buf)
                m1 = jnp.where(v1 == fE[e], 1.0, 0.0)
                cs1 = _iprefix(m1, buf)
                r1 = cs1 - m1 + carry1[e]
                pos1c = pos1c + ((st1b[e] + r1.astype(jnp.int32))
                                 * m1.astype(jnp.int32))
                carry1[e] = carry1[e] + _lane_bcast(cs1, LANES - 1, iota, buf)
            p0_v[pl.ds(c * LANES, LANES)] = pos0c
            p1_v[pl.ds(c * LANES, LANES)] = pos1c

        pltpu.sync_copy(p0_v, pos0_hbm.at[pl.ds(base, tpw)])
        pltpu.sync_copy(p1_v, pos1_hbm.at[pl.ds(base, tpw)])

        cp.wait()
        c0 = pltpu.async_copy(xr_v, xp_hbm.at[p0_v], s0)
        c0.wait()
        c1 = pltpu.async_copy(xr_v, xp_hbm.at[p1_v], s0)
        c1.wait()
        w0cp = pltpu.async_copy(w0_v, wp_hbm.at[p0_v], s1)
        w0cp.wait()
        w1cp = pltpu.async_copy(w1_v, wp_hbm.at[p1_v], s1)
        w1cp.wait()

    return k(x, e0f, e1f, w0c, w1c)


def _moe_mm(tile_e, x_perm, w1b, w2b, w_col):
    def body(te_ref, x_ref, w1_ref, w2_ref, ws_ref, o_ref):
        xb = x_ref[...].astype(jnp.bfloat16)
        h = jnp.dot(xb, w1_ref[0], preferred_element_type=jnp.float32)
        h = h * jax.nn.sigmoid(h)
        y = jnp.dot(h.astype(jnp.bfloat16), w2_ref[0],
                    preferred_element_type=jnp.float32)
        o_ref[...] = y * ws_ref[:, 0:1]

    grid_spec = pltpu.PrefetchScalarGridSpec(
        num_scalar_prefetch=1,
        grid=(NTILES,),
        in_specs=[
            pl.BlockSpec((BM, H_), lambda i, te: (i, 0)),
            pl.BlockSpec((1, H_, I_), lambda i, te: (te[i], 0, 0)),
            pl.BlockSpec((1, I_, H_), lambda i, te: (te[i], 0, 0)),
            pl.BlockSpec((BM, _EPAD), lambda i, te: (i, 0)),
        ],
        out_specs=pl.BlockSpec((BM, H_), lambda i, te: (i, 0)),
    )
    return pl.pallas_call(
        body,
        grid_spec=grid_spec,
        out_shape=jax.ShapeDtypeStruct((PPAD, H_), jnp.float32),
        compiler_params=pltpu.CompilerParams(
            dimension_semantics=("arbitrary",)),
    )(tile_e, x_perm, w1b, w2b, w_col)


def _sc_combine(y_perm, pos0, pos1):
    tpw = T_ // NW
    ch = 32
    mesh = plsc.VectorSubcoreMesh(core_axis_name="c", subcore_axis_name="s",
                                  num_cores=NC, num_subcores=NS)

    @functools.partial(
        pl.kernel,
        out_type=jax.ShapeDtypeStruct((T_, H_), jnp.float32),
        mesh=mesh,
        scratch_types=[
            pltpu.VMEM((ch,), jnp.int32),
            pltpu.VMEM((ch,), jnp.int32),
            pltpu.VMEM((ch, H_), jnp.float32),
            pltpu.VMEM((ch, H_), jnp.float32),
            pltpu.SemaphoreType.DMA,
            pltpu.SemaphoreType.DMA,
        ],
    )
    def k(y_hbm, p0_hbm, p1_hbm, out_hbm, i0_v, i1_v, r0_v, r1_v, s0, s1):
        wid = lax.axis_index("s") * NC + lax.axis_index("c")
        base = wid * tpw
        for c in range(tpw // ch):
            off = base + c * ch
            pltpu.sync_copy(p0_hbm.at[pl.ds(off, ch)], i0_v)
            pltpu.sync_copy(p1_hbm.at[pl.ds(off, ch)], i1_v)
            cp0 = pltpu.async_copy(y_hbm.at[i0_v], r0_v, s0)
            cp1 = pltpu.async_copy(y_hbm.at[i1_v], r1_v, s1)
            cp0.wait()
            cp1.wait()

            def addrow(t, _):
                for kk in range(H_ // LANES):
                    sl = pl.ds(kk * LANES, LANES)
                    r0_v[t, sl] = r0_v[t, sl] + r1_v[t, sl]
                return 0

            lax.fori_loop(0, ch, addrow, 0)
            pltpu.sync_copy(r0_v, out_hbm.at[pl.ds(off, ch)])

    return k(y_perm, pos0, pos1)


def kernel(hidden_states, gate_w, experts_w1, experts_w2):
    router_logits = hidden_states @ gate_w
    probs = jax.nn.softmax(router_logits.astype(jnp.float32), axis=-1)
    probs_pad = jnp.full((T_, _EPAD), -1.0, jnp.float32).at[:, :E_].set(probs)
    i0f, i1f, w0, w1 = _router(probs_pad)
    x_perm, w_perm, pos0, pos1, tile_e = _sc_dispatch(
        hidden_states, i0f.reshape(T_), i1f.reshape(T_), w0, w1)
    w1b = experts_w1.astype(jnp.bfloat16)
    w2b = experts_w2.astype(jnp.bfloat16)
    y_perm = _moe_mm(tile_e[:NTILES], x_perm, w1b, w2b, w_perm)
    return _sc_combine(y_perm, pos0, pos1)


# speedup vs baseline: 1.0951x; 1.0951x over previous
import functools

import jax
import jax.numpy as jnp
from jax import lax
from jax.experimental import pallas as pl
from jax.experimental.pallas import tpu as pltpu
from jax.experimental.pallas import tpu_sc as plsc

T_ = 2048
H_ = 1024
E_ = 8
I_ = 1408
K_ = 2

BM = 256
PPAD = T_ * K_ + E_ * BM
NTILES = PPAD // BM

NC = 2
NS = 16
NW = NC * NS
LANES = 16

_EPAD = 128


def _router(probs_pad):
    BT = 256

    def body(p_ref, i0_ref, i1_ref, w0_ref, w1_ref):
        p = p_ref[...]
        cols = lax.broadcasted_iota(jnp.int32, (BT, _EPAD), 1)
        m1 = jnp.max(p, axis=1, keepdims=True)
        i1 = jnp.min(jnp.where(p == m1, cols, _EPAD), axis=1, keepdims=True)
        p2 = jnp.where(cols == i1, jnp.float32(-2.0), p)
        m2 = jnp.max(p2, axis=1, keepdims=True)
        i2 = jnp.min(jnp.where(p2 == m2, cols, _EPAD), axis=1, keepdims=True)
        s = m1 + m2
        ones = jnp.ones((BT, _EPAD), jnp.float32)
        i0_ref[...] = i1.astype(jnp.float32)
        i1_ref[...] = i2.astype(jnp.float32)
        w0_ref[...] = (m1 / s) * ones
        w1_ref[...] = (m2 / s) * ones

    return pl.pallas_call(
        body,
        grid=(T_ // BT,),
        in_specs=[pl.BlockSpec((BT, _EPAD), lambda i: (i, 0))],
        out_specs=[pl.BlockSpec((BT, 1), lambda i: (i, 0))] * 2
        + [pl.BlockSpec((BT, _EPAD), lambda i: (i, 0))] * 2,
        out_shape=[jax.ShapeDtypeStruct((T_, 1), jnp.float32)] * 2
        + [jax.ShapeDtypeStruct((T_, _EPAD), jnp.float32)] * 2,
    )(probs_pad)


def _allsum(v, buf):
    t = v
    for s in (1, 2, 4, 8):
        buf[pl.ds(0, LANES)] = t
        buf[pl.ds(LANES, LANES)] = t
        t = t + buf[pl.ds(s, LANES)]
    return t


def _iprefix(v, buf):
    buf[pl.ds(0, LANES)] = jnp.zeros_like(v)
    p = v
    for s in (1, 2, 4, 8):
        buf[pl.ds(LANES, LANES)] = p
        p = p + buf[pl.ds(LANES - s, LANES)]
    return p


def _lane_bcast(v, e, iota, buf):
    sel = jnp.where(iota == e, v, jnp.zeros_like(v))
    return _allsum(sel, buf)


def _sc_dispatch(x, e0f, e1f, w0c, w1c):
    tpw = T_ // NW
    nch = tpw // LANES
    mesh = plsc.VectorSubcoreMesh(core_axis_name="c", subcore_axis_name="s",
                                  num_cores=NC, num_subcores=NS)
    fE = [jnp.float32(e) for e in range(E_)]

    @functools.partial(
        pl.kernel,
        out_type=[
            jax.ShapeDtypeStruct((PPAD, H_), jnp.float32),
            jax.ShapeDtypeStruct((PPAD, _EPAD), jnp.float32),
            jax.ShapeDtypeStruct((T_,), jnp.int32),
            jax.ShapeDtypeStruct((T_,), jnp.int32),
            jax.ShapeDtypeStruct((32,), jnp.int32),
        ],
        mesh=mesh,
        scratch_types=[
            pltpu.VMEM((tpw, H_), jnp.float32),
            pltpu.VMEM((T_,), jnp.float32),
            pltpu.VMEM((T_,), jnp.float32),
            pltpu.VMEM((tpw, _EPAD), jnp.float32),
            pltpu.VMEM((tpw, _EPAD), jnp.float32),
            pltpu.VMEM((tpw,), jnp.int32),
            pltpu.VMEM((tpw,), jnp.int32),
            pltpu.VMEM((32,), jnp.int32),
            pltpu.VMEM((2 * LANES,), jnp.float32),
            pltpu.VMEM((2 * LANES,), jnp.int32),
            pltpu.SemaphoreType.DMA,
            pltpu.SemaphoreType.DMA,
        ],
    )
    def k(x_hbm, e0_hbm, e1_hbm, w0_hbm, w1_hbm,
          xp_hbm, wp_hbm, pos0_hbm, pos1_hbm, te_hbm,
          xr_v, ge0_v, ge1_v, w0_v, w1_v, p0_v, p1_v, tile_v,
          buf, bufi, s0, s1):
        wid = lax.axis_index("s") * NC + lax.axis_index("c")
        base = wid * tpw

        cp = pltpu.async_copy(x_hbm.at[pl.ds(base, tpw)], xr_v, s1)
        pltpu.sync_copy(e0_hbm, ge0_v)
        pltpu.sync_copy(e1_hbm, ge1_v)
        pltpu.sync_copy(w0_hbm.at[pl.ds(base, tpw)], w0_v)
        pltpu.sync_copy(w1_hbm.at[pl.ds(base, tpw)], w1_v)

        iota = jnp.arange(LANES, dtype=jnp.int32)
        zero = jnp.zeros((LANES,), jnp.float32)
        zi = jnp.zeros((LANES,), jnp.int32)

        def scan_body(i, carry):
            v0 = ge0_v[pl.ds(i * LANES, LANES)]
            v1 = ge1_v[pl.ds(i * LANES, LANES)]
            pre = jnp.where((zi + i * LANES) < base, 1.0, 0.0)
            new = []
            for e in range(E_):
                c0, p0, c1, p1 = carry[4 * e:4 * e + 4]
                m0 = jnp.where(v0 == fE[e], 1.0, 0.0)
                m1 = jnp.where(v1 == fE[e], 1.0, 0.0)
                new.extend([c0 + m0, p0 + m0 * pre, c1 + m1, p1 + m1 * pre])
            return tuple(new)

        accs = lax.fori_loop(0, T_ // LANES, scan_body, tuple([zero] * (4 * E_)))

        cnt_vec = zi
        c0b, p0b, p1b = [], [], []
        for e in range(E_):
            c0 = _allsum(accs[4 * e], buf).astype(jnp.int32)
            p0 = _allsum(accs[4 * e + 1], buf).astype(jnp.int32)
            c1 = _allsum(accs[4 * e + 2], buf).astype(jnp.int32)
            p1 = _allsum(accs[4 * e + 3], buf).astype(jnp.int32)
            c0b.append(c0)
            p0b.append(p0)
            p1b.append(p1)
            cnt_vec = jnp.where(iota == e, c0 + c1, cnt_vec)

        cap_vec = (cnt_vec + (BM - 1)) & (-BM)
        bufi[pl.ds(0, LANES)] = zi
        ends_vec = _iprefix(cap_vec, bufi)
        offs_vec = ends_vec - cap_vec
        st0b, st1b = [], []
        for e in range(E_):
            ob = _lane_bcast(offs_vec, e, iota, bufi)
            st0b.append(ob + p0b[e])
            st1b.append(ob + c0b[e] + p1b[e])

        @pl.when(wid == 0)
        def _():
            for c in range(32 // LANES):
                tstart = (iota + c * LANES) * BM
                te_vec = zi
                for e in range(E_ - 1):
                    eb = _lane_bcast(ends_vec, e, iota, bufi)
                    te_vec = te_vec + jnp.where(tstart >= eb, 1, 0)
                tile_v[pl.ds(c * LANES, LANES)] = te_vec
            pltpu.sync_copy(tile_v, te_hbm)

        carry0 = [zero] * E_
        carry1 = [zero] * E_
        for c in range(nch):
            v0 = ge0_v[pl.ds(base + c * LANES, LANES)]
            v1 = ge1_v[pl.ds(base + c * LANES, LANES)]
            pos0c = zi
            pos1c = zi
            for e in range(E_):
                m0 = jnp.where(v0 == fE[e], 1.0, 0.0)
                cs0 = _iprefix(m0, buf)
                r0 = cs0 - m0 + carry0[e]
                pos0c = pos0c + ((st0b[e] + r0.astype(jnp.int32))
                                 * m0.astype(jnp.int32))
                carry0[e] = carry0[e] + _lane_bcast(cs0, LANES - 1, iota, buf)
                m1 = jnp.where(v1 == fE[e], 1.0, 0.0)
                cs1 = _iprefix(m1, buf)
                r1 = cs1 - m1 + carry1[e]
                pos1c = pos1c + ((st1b[e] + r1.astype(jnp.int32))
                                 * m1.astype(jnp.int32))
                carry1[e] = carry1[e] + _lane_bcast(cs1, LANES - 1, iota, buf)
            p0_v[pl.ds(c * LANES, LANES)] = pos0c
            p1_v[pl.ds(c * LANES, LANES)] = pos1c

        pltpu.sync_copy(p0_v, pos0_hbm.at[pl.ds(base, tpw)])
        pltpu.sync_copy(p1_v, pos1_hbm.at[pl.ds(base, tpw)])

        cp.wait()
        c0 = pltpu.async_copy(xr_v, xp_hbm.at[p0_v], s0)
        c0.wait()
        c1 = pltpu.async_copy(xr_v, xp_hbm.at[p1_v], s0)
        c1.wait()
        w0cp = pltpu.async_copy(w0_v, wp_hbm.at[p0_v], s1)
        w0cp.wait()
        w1cp = pltpu.async_copy(w1_v, wp_hbm.at[p1_v], s1)
        w1cp.wait()

    return k(x, e0f, e1f, w0c, w1c)


def _moe_mm(tile_e, x_perm, w1b, w2b, w_col):
    def body(te_ref, x_ref, w1_ref, w2_ref, ws_ref, o_ref):
        h = jnp.dot(x_ref[...], w1_ref[0], preferred_element_type=jnp.float32)
        h = h * jax.nn.sigmoid(h)
        y = jnp.dot(h, w2_ref[0], preferred_element_type=jnp.float32)
        o_ref[...] = y * ws_ref[:, 0:1]

    grid_spec = pltpu.PrefetchScalarGridSpec(
        num_scalar_prefetch=1,
        grid=(NTILES,),
        in_specs=[
            pl.BlockSpec((BM, H_), lambda i, te: (i, 0)),
            pl.BlockSpec((1, H_, I_), lambda i, te: (te[i], 0, 0)),
            pl.BlockSpec((1, I_, H_), lambda i, te: (te[i], 0, 0)),
            pl.BlockSpec((BM, _EPAD), lambda i, te: (i, 0)),
        ],
        out_specs=pl.BlockSpec((BM, H_), lambda i, te: (i, 0)),
    )
    return pl.pallas_call(
        body,
        grid_spec=grid_spec,
        out_shape=jax.ShapeDtypeStruct((PPAD, H_), jnp.float32),
        compiler_params=pltpu.CompilerParams(
            dimension_semantics=("arbitrary",)),
    )(tile_e, x_perm, w1b, w2b, w_col)


def _sc_combine(y_perm, pos0, pos1):
    tpw = T_ // NW
    ch = 32
    mesh = plsc.VectorSubcoreMesh(core_axis_name="c", subcore_axis_name="s",
                                  num_cores=NC, num_subcores=NS)

    @functools.partial(
        pl.kernel,
        out_type=jax.ShapeDtypeStruct((T_, H_), jnp.float32),
        mesh=mesh,
        scratch_types=[
            pltpu.VMEM((ch,), jnp.int32),
            pltpu.VMEM((ch,), jnp.int32),
            pltpu.VMEM((ch, H_), jnp.float32),
            pltpu.VMEM((ch, H_), jnp.float32),
            pltpu.SemaphoreType.DMA,
            pltpu.SemaphoreType.DMA,
        ],
    )
    def k(y_hbm, p0_hbm, p1_hbm, out_hbm, i0_v, i1_v, r0_v, r1_v, s0, s1):
        wid = lax.axis_index("s") * NC + lax.axis_index("c")
        base = wid * tpw
        for c in range(tpw // ch):
            off = base + c * ch
            pltpu.sync_copy(p0_hbm.at[pl.ds(off, ch)], i0_v)
            pltpu.sync_copy(p1_hbm.at[pl.ds(off, ch)], i1_v)
            cp0 = pltpu.async_copy(y_hbm.at[i0_v], r0_v, s0)
            cp1 = pltpu.async_copy(y_hbm.at[i1_v], r1_v, s1)
            cp0.wait()
            cp1.wait()

            def addrow(t, _):
                for kk in range(H_ // LANES):
                    sl = pl.ds(kk * LANES, LANES)
                    r0_v[t, sl] = r0_v[t, sl] + r1_v[t, sl]
                return 0

            lax.fori_loop(0, ch, addrow, 0)
            pltpu.sync_copy(r0_v, out_hbm.at[pl.ds(off, ch)])

    return k(y_perm, pos0, pos1)


def kernel(hidden_states, gate_w, experts_w1, experts_w2):
    router_logits = hidden_states @ gate_w
    probs = jax.nn.softmax(router_logits.astype(jnp.float32), axis=-1)
    probs_pad = jnp.full((T_, _EPAD), -1.0, jnp.float32).at[:, :E_].set(probs)
    i0f, i1f, w0, w1 = _router(probs_pad)
    x_perm, w_perm, pos0, pos1, tile_e = _sc_dispatch(
        hidden_states, i0f.reshape(T_), i1f.reshape(T_), w0, w1)
    y_perm = _moe_mm(tile_e[:NTILES], x_perm, experts_w1, experts_w2, w_perm)
    return _sc_combine(y_perm, pos0, pos1)


# speedup vs baseline: 1.1086x; 1.0122x over previous
import functools

import jax
import jax.numpy as jnp
from jax import lax
from jax.experimental import pallas as pl
from jax.experimental.pallas import tpu as pltpu
from jax.experimental.pallas import tpu_sc as plsc

T_ = 2048
H_ = 1024
E_ = 8
I_ = 1408
K_ = 2

BM = 256
PPAD = T_ * K_ + E_ * BM
NTILES = PPAD // BM

NC = 2
NS = 16
NW = NC * NS
LANES = 16

_EPAD = 128


def _router(probs):
    BT = 256

    def body(p_ref, i0_ref, i1_ref, w0_ref, w1_ref):
        p = p_ref[...]
        cols = lax.broadcasted_iota(jnp.int32, (BT, E_), 1)
        m1 = jnp.max(p, axis=1, keepdims=True)
        i1 = jnp.min(jnp.where(p == m1, cols, E_), axis=1, keepdims=True)
        p2 = jnp.where(cols == i1, jnp.float32(-2.0), p)
        m2 = jnp.max(p2, axis=1, keepdims=True)
        i2 = jnp.min(jnp.where(p2 == m2, cols, E_), axis=1, keepdims=True)
        s = m1 + m2
        ones = jnp.ones((BT, _EPAD), jnp.float32)
        i0_ref[...] = i1.astype(jnp.float32)
        i1_ref[...] = i2.astype(jnp.float32)
        w0_ref[...] = (m1 / s) * ones
        w1_ref[...] = (m2 / s) * ones

    return pl.pallas_call(
        body,
        grid=(T_ // BT,),
        in_specs=[pl.BlockSpec((BT, E_), lambda i: (i, 0))],
        out_specs=[pl.BlockSpec((BT, 1), lambda i: (i, 0))] * 2
        + [pl.BlockSpec((BT, _EPAD), lambda i: (i, 0))] * 2,
        out_shape=[jax.ShapeDtypeStruct((T_, 1), jnp.float32)] * 2
        + [jax.ShapeDtypeStruct((T_, _EPAD), jnp.float32)] * 2,
    )(probs)


def _allsum(v, buf):
    t = v
    for s in (1, 2, 4, 8):
        buf[pl.ds(0, LANES)] = t
        buf[pl.ds(LANES, LANES)] = t
        t = t + buf[pl.ds(s, LANES)]
    return t


def _iprefix(v, buf):
    buf[pl.ds(0, LANES)] = jnp.zeros_like(v)
    p = v
    for s in (1, 2, 4, 8):
        buf[pl.ds(LANES, LANES)] = p
        p = p + buf[pl.ds(LANES - s, LANES)]
    return p


def _lane_bcast(v, e, iota, buf):
    sel = jnp.where(iota == e, v, jnp.zeros_like(v))
    return _allsum(sel, buf)


def _sc_dispatch(x, e0f, e1f, w0c, w1c):
    tpw = T_ // NW
    nch = tpw // LANES
    mesh = plsc.VectorSubcoreMesh(core_axis_name="c", subcore_axis_name="s",
                                  num_cores=NC, num_subcores=NS)
    fE = [jnp.float32(e) for e in range(E_)]

    @functools.partial(
        pl.kernel,
        out_type=[
            jax.ShapeDtypeStruct((PPAD, H_), jnp.float32),
            jax.ShapeDtypeStruct((PPAD, _EPAD), jnp.float32),
            jax.ShapeDtypeStruct((T_,), jnp.int32),
            jax.ShapeDtypeStruct((T_,), jnp.int32),
            jax.ShapeDtypeStruct((32,), jnp.int32),
        ],
        mesh=mesh,
        scratch_types=[
            pltpu.VMEM((tpw, H_), jnp.float32),
            pltpu.VMEM((T_,), jnp.float32),
            pltpu.VMEM((T_,), jnp.float32),
            pltpu.VMEM((tpw, _EPAD), jnp.float32),
            pltpu.VMEM((tpw, _EPAD), jnp.float32),
            pltpu.VMEM((tpw,), jnp.int32),
            pltpu.VMEM((tpw,), jnp.int32),
            pltpu.VMEM((32,), jnp.int32),
            pltpu.VMEM((2 * LANES,), jnp.float32),
            pltpu.VMEM((2 * LANES,), jnp.int32),
            pltpu.SemaphoreType.DMA,
            pltpu.SemaphoreType.DMA,
        ],
    )
    def k(x_hbm, e0_hbm, e1_hbm, w0_hbm, w1_hbm,
          xp_hbm, wp_hbm, pos0_hbm, pos1_hbm, te_hbm,
          xr_v, ge0_v, ge1_v, w0_v, w1_v, p0_v, p1_v, tile_v,
          buf, bufi, s0, s1):
        wid = lax.axis_index("s") * NC + lax.axis_index("c")
        base = wid * tpw

        cp = pltpu.async_copy(x_hbm.at[pl.ds(base, tpw)], xr_v, s1)
        pltpu.sync_copy(e0_hbm, ge0_v)
        pltpu.sync_copy(e1_hbm, ge1_v)
        pltpu.sync_copy(w0_hbm.at[pl.ds(base, tpw)], w0_v)
        pltpu.sync_copy(w1_hbm.at[pl.ds(base, tpw)], w1_v)

        iota = jnp.arange(LANES, dtype=jnp.int32)
        zero = jnp.zeros((LANES,), jnp.float32)
        zi = jnp.zeros((LANES,), jnp.int32)

        def scan_body(i, carry):
            v0 = ge0_v[pl.ds(i * LANES, LANES)]
            v1 = ge1_v[pl.ds(i * LANES, LANES)]
            pre = jnp.where((zi + i * LANES) < base, 1.0, 0.0)
            new = []
            for e in range(E_):
                c0, p0, c1, p1 = carry[4 * e:4 * e + 4]
                m0 = jnp.where(v0 == fE[e], 1.0, 0.0)
                m1 = jnp.where(v1 == fE[e], 1.0, 0.0)
                new.extend([c0 + m0, p0 + m0 * pre, c1 + m1, p1 + m1 * pre])
            return tuple(new)

        accs = lax.fori_loop(0, T_ // LANES, scan_body, tuple([zero] * (4 * E_)))

        cnt_vec = zi
        c0b, p0b, p1b = [], [], []
        for e in range(E_):
            c0 = _allsum(accs[4 * e], buf).astype(jnp.int32)
            p0 = _allsum(accs[4 * e + 1], buf).astype(jnp.int32)
            c1 = _allsum(accs[4 * e + 2], buf).astype(jnp.int32)
            p1 = _allsum(accs[4 * e + 3], buf).astype(jnp.int32)
            c0b.append(c0)
            p0b.append(p0)
            p1b.append(p1)
            cnt_vec = jnp.where(iota == e, c0 + c1, cnt_vec)

        cap_vec = (cnt_vec + (BM - 1)) & (-BM)
        bufi[pl.ds(0, LANES)] = zi
        ends_vec = _iprefix(cap_vec, bufi)
        offs_vec = ends_vec - cap_vec
        st0b, st1b = [], []
        for e in range(E_):
            ob = _lane_bcast(offs_vec, e, iota, bufi)
            st0b.append(ob + p0b[e])
            st1b.append(ob + c0b[e] + p1b[e])

        @pl.when(wid == 0)
        def _():
            for c in range(32 // LANES):
                tstart = (iota + c * LANES) * BM
                te_vec = zi
                for e in range(E_ - 1):
                    eb = _lane_bcast(ends_vec, e, iota, bufi)
                    te_vec = te_vec + jnp.where(tstart >= eb, 1, 0)
                tile_v[pl.ds(c * LANES, LANES)] = te_vec
            pltpu.sync_copy(tile_v, te_hbm)

        carry0 = [zero] * E_
        carry1 = [zero] * E_
        for c in range(nch):
            v0 = ge0_v[pl.ds(base + c * LANES, LANES)]
            v1 = ge1_v[pl.ds(base + c * LANES, LANES)]
            pos0c = zi
            pos1c = zi
            for e in range(E_):
                m0 = jnp.where(v0 == fE[e], 1.0, 0.0)
                cs0 = _iprefix(m0, buf)
                r0 = cs0 - m0 + carry0[e]
                pos0c = pos0c + ((st0b[e] + r0.astype(jnp.int32))
                                 * m0.astype(jnp.int32))
                carry0[e] = carry0[e] + _lane_bcast(cs0, LANES - 1, iota, buf)
                m1 = jnp.where(v1 == fE[e], 1.0, 0.0)
                cs1 = _iprefix(m1, buf)
                r1 = cs1 - m1 + carry1[e]
                pos1c = pos1c + ((st1b[e] + r1.astype(jnp.int32))
                                 * m1.astype(jnp.int32))
                carry1[e] = carry1[e] + _lane_bcast(cs1, LANES - 1, iota, buf)
            p0_v[pl.ds(c * LANES, LANES)] = pos0c
            p1_v[pl.ds(c * LANES, LANES)] = pos1c

        pltpu.sync_copy(p0_v, pos0_hbm.at[pl.ds(base, tpw)])
        pltpu.sync_copy(p1_v, pos1_hbm.at[pl.ds(base, tpw)])

        cp.wait()
        c0 = pltpu.async_copy(xr_v, xp_hbm.at[p0_v], s0)
        c1 = pltpu.async_copy(xr_v, xp_hbm.at[p1_v], s0)
        w0cp = pltpu.async_copy(w0_v, wp_hbm.at[p0_v], s1)
        w1cp = pltpu.async_copy(w1_v, wp_hbm.at[p1_v], s1)
        c0.wait()
        c1.wait()
        w0cp.wait()
        w1cp.wait()

    return k(x, e0f, e1f, w0c, w1c)


def _moe_mm(tile_e, x_perm, w1b, w2b, w_col):
    def body(te_ref, x_ref, w1_ref, w2_ref, ws_ref, o_ref):
        h = jnp.dot(x_ref[...], w1_ref[0], preferred_element_type=jnp.float32)
        h = h * jax.nn.sigmoid(h)
        y = jnp.dot(h, w2_ref[0], preferred_element_type=jnp.float32)
        o_ref[...] = y * ws_ref[:, 0:1]

    grid_spec = pltpu.PrefetchScalarGridSpec(
        num_scalar_prefetch=1,
        grid=(NTILES,),
        in_specs=[
            pl.BlockSpec((BM, H_), lambda i, te: (i, 0)),
            pl.BlockSpec((1, H_, I_), lambda i, te: (te[i], 0, 0)),
            pl.BlockSpec((1, I_, H_), lambda i, te: (te[i], 0, 0)),
            pl.BlockSpec((BM, _EPAD), lambda i, te: (i, 0)),
        ],
        out_specs=pl.BlockSpec((BM, H_), lambda i, te: (i, 0)),
    )
    return pl.pallas_call(
        body,
        grid_spec=grid_spec,
        out_shape=jax.ShapeDtypeStruct((PPAD, H_), jnp.float32),
        compiler_params=pltpu.CompilerParams(
            dimension_semantics=("arbitrary",)),
    )(tile_e, x_perm, w1b, w2b, w_col)


def _sc_combine(y_perm, pos0, pos1):
    tpw = T_ // NW
    ch = 32
    mesh = plsc.VectorSubcoreMesh(core_axis_name="c", subcore_axis_name="s",
                                  num_cores=NC, num_subcores=NS)

    @functools.partial(
        pl.kernel,
        out_type=jax.ShapeDtypeStruct((T_, H_), jnp.float32),
        mesh=mesh,
        scratch_types=[
            pltpu.VMEM((ch,), jnp.int32),
            pltpu.VMEM((ch,), jnp.int32),
            pltpu.VMEM((ch, H_), jnp.float32),
            pltpu.VMEM((ch, H_), jnp.float32),
            pltpu.SemaphoreType.DMA,
            pltpu.SemaphoreType.DMA,
        ],
    )
    def k(y_hbm, p0_hbm, p1_hbm, out_hbm, i0_v, i1_v, r0_v, r1_v, s0, s1):
        wid = lax.axis_index("s") * NC + lax.axis_index("c")
        base = wid * tpw
        for c in range(tpw // ch):
            off = base + c * ch
            pltpu.sync_copy(p0_hbm.at[pl.ds(off, ch)], i0_v)
            pltpu.sync_copy(p1_hbm.at[pl.ds(off, ch)], i1_v)
            cp0 = pltpu.async_copy(y_hbm.at[i0_v], r0_v, s0)
            cp1 = pltpu.async_copy(y_hbm.at[i1_v], r1_v, s1)
            cp0.wait()
            cp1.wait()

            def addrow(t, _):
                for kk in range(H_ // LANES):
                    sl = pl.ds(kk * LANES, LANES)
                    r0_v[t, sl] = r0_v[t, sl] + r1_v[t, sl]
                return 0

            lax.fori_loop(0, ch, addrow, 0)
            pltpu.sync_copy(r0_v, out_hbm.at[pl.ds(off, ch)])

    return k(y_perm, pos0, pos1)


def kernel(hidden_states, gate_w, experts_w1, experts_w2):
    router_logits = hidden_states @ gate_w
    probs = jax.nn.softmax(router_logits.astype(jnp.float32), axis=-1)
    i0f, i1f, w0, w1 = _router(probs)
    x_perm, w_perm, pos0, pos1, tile_e = _sc_dispatch(
        hidden_states, i0f.reshape(T_), i1f.reshape(T_), w0, w1)
    y_perm = _moe_mm(tile_e[:NTILES], x_perm, experts_w1, experts_w2, w_perm)
    return _sc_combine(y_perm, pos0, pos1)


# speedup vs baseline: 1.1209x; 1.0111x over previous
import functools

import jax
import jax.numpy as jnp
from jax import lax
from jax.experimental import pallas as pl
from jax.experimental.pallas import tpu as pltpu
from jax.experimental.pallas import tpu_sc as plsc

T_ = 2048
H_ = 1024
E_ = 8
I_ = 1408
K_ = 2

BM = 256
PPAD = T_ * K_ + E_ * BM
NTILES = PPAD // BM

NC = 2
NS = 16
NW = NC * NS
LANES = 16

_EPAD = 128


def _router(probs):
    BT = 256

    def body(p_ref, i0_ref, i1_ref, w0_ref, w1_ref):
        p = p_ref[...]
        cols = lax.broadcasted_iota(jnp.int32, (BT, E_), 1)
        m1 = jnp.max(p, axis=1, keepdims=True)
        i1 = jnp.min(jnp.where(p == m1, cols, E_), axis=1, keepdims=True)
        p2 = jnp.where(cols == i1, jnp.float32(-2.0), p)
        m2 = jnp.max(p2, axis=1, keepdims=True)
        i2 = jnp.min(jnp.where(p2 == m2, cols, E_), axis=1, keepdims=True)
        s = m1 + m2
        ones = jnp.ones((BT, _EPAD), jnp.float32)
        i0_ref[...] = i1.astype(jnp.float32)
        i1_ref[...] = i2.astype(jnp.float32)
        w0_ref[...] = (m1 / s) * ones
        w1_ref[...] = (m2 / s) * ones

    return pl.pallas_call(
        body,
        grid=(T_ // BT,),
        in_specs=[pl.BlockSpec((BT, E_), lambda i: (i, 0))],
        out_specs=[pl.BlockSpec((BT, 1), lambda i: (i, 0))] * 2
        + [pl.BlockSpec((BT, _EPAD), lambda i: (i, 0))] * 2,
        out_shape=[jax.ShapeDtypeStruct((T_, 1), jnp.float32)] * 2
        + [jax.ShapeDtypeStruct((T_, _EPAD), jnp.float32)] * 2,
    )(probs)


def _allsum(v, buf):
    t = v
    for s in (1, 2, 4, 8):
        buf[pl.ds(0, LANES)] = t
        buf[pl.ds(LANES, LANES)] = t
        t = t + buf[pl.ds(s, LANES)]
    return t


def _iprefix(v, buf):
    buf[pl.ds(0, LANES)] = jnp.zeros_like(v)
    p = v
    for s in (1, 2, 4, 8):
        buf[pl.ds(LANES, LANES)] = p
        p = p + buf[pl.ds(LANES - s, LANES)]
    return p


def _lane_bcast(v, e, iota, buf):
    sel = jnp.where(iota == e, v, jnp.zeros_like(v))
    return _allsum(sel, buf)


def _sc_dispatch(x, e0f, e1f, w0c, w1c):
    tpw = T_ // NW
    nch = tpw // LANES
    mesh = plsc.VectorSubcoreMesh(core_axis_name="c", subcore_axis_name="s",
                                  num_cores=NC, num_subcores=NS)
    fE = [jnp.float32(e) for e in range(E_)]

    @functools.partial(
        pl.kernel,
        out_type=[
            jax.ShapeDtypeStruct((PPAD, H_), jnp.float32),
            jax.ShapeDtypeStruct((PPAD, _EPAD), jnp.float32),
            jax.ShapeDtypeStruct((T_,), jnp.int32),
            jax.ShapeDtypeStruct((T_,), jnp.int32),
            jax.ShapeDtypeStruct((32,), jnp.int32),
        ],
        mesh=mesh,
        scratch_types=[
            pltpu.VMEM((tpw, H_), jnp.float32),
            pltpu.VMEM((T_,), jnp.float32),
            pltpu.VMEM((T_,), jnp.float32),
            pltpu.VMEM((tpw, _EPAD), jnp.float32),
            pltpu.VMEM((tpw, _EPAD), jnp.float32),
            pltpu.VMEM((tpw,), jnp.int32),
            pltpu.VMEM((tpw,), jnp.int32),
            pltpu.VMEM((32,), jnp.int32),
            pltpu.VMEM((2 * LANES,), jnp.float32),
            pltpu.VMEM((2 * LANES,), jnp.int32),
            pltpu.SemaphoreType.DMA,
            pltpu.SemaphoreType.DMA,
        ],
    )
    def k(x_hbm, e0_hbm, e1_hbm, w0_hbm, w1_hbm,
          xp_hbm, wp_hbm, pos0_hbm, pos1_hbm, te_hbm,
          xr_v, ge0_v, ge1_v, w0_v, w1_v, p0_v, p1_v, tile_v,
          buf, bufi, s0, s1):
        wid = lax.axis_index("s") * NC + lax.axis_index("c")
        base = wid * tpw

        cp = pltpu.async_copy(x_hbm.at[pl.ds(base, tpw)], xr_v, s1)
        pltpu.sync_copy(e0_hbm, ge0_v)
        pltpu.sync_copy(e1_hbm, ge1_v)
        pltpu.sync_copy(w0_hbm.at[pl.ds(base, tpw)], w0_v)
        pltpu.sync_copy(w1_hbm.at[pl.ds(base, tpw)], w1_v)

        iota = jnp.arange(LANES, dtype=jnp.int32)
        zero = jnp.zeros((LANES,), jnp.float32)
        zi = jnp.zeros((LANES,), jnp.int32)

        def scan_body(i, carry):
            v0 = ge0_v[pl.ds(i * LANES, LANES)]
            v1 = ge1_v[pl.ds(i * LANES, LANES)]
            pre = jnp.where((zi + i * LANES) < base, 1.0, 0.0)
            new = []
            for e in range(E_):
                c0, p0, c1, p1 = carry[4 * e:4 * e + 4]
                m0 = jnp.where(v0 == fE[e], 1.0, 0.0)
                m1 = jnp.where(v1 == fE[e], 1.0, 0.0)
                new.extend([c0 + m0, p0 + m0 * pre, c1 + m1, p1 + m1 * pre])
            return tuple(new)

        accs = lax.fori_loop(0, T_ // LANES, scan_body, tuple([zero] * (4 * E_)))

        cnt_vec = zi
        c0b, p0b, p1b = [], [], []
        for e in range(E_):
            c0 = _allsum(accs[4 * e], buf).astype(jnp.int32)
            p0 = _allsum(accs[4 * e + 1], buf).astype(jnp.int32)
            c1 = _allsum(accs[4 * e + 2], buf).astype(jnp.int32)
            p1 = _allsum(accs[4 * e + 3], buf).astype(jnp.int32)
            c0b.append(c0)
            p0b.append(p0)
            p1b.append(p1)
            cnt_vec = jnp.where(iota == e, c0 + c1, cnt_vec)

        cap_vec = (cnt_vec + (BM - 1)) & (-BM)
        bufi[pl.ds(0, LANES)] = zi
        ends_vec = _iprefix(cap_vec, bufi)
        offs_vec = ends_vec - cap_vec
        st0b, st1b = [], []
        for e in range(E_):
            ob = _lane_bcast(offs_vec, e, iota, bufi)
            st0b.append(ob + p0b[e])
            st1b.append(ob + c0b[e] + p1b[e])

        @pl.when(wid == 0)
        def _():
            for c in range(32 // LANES):
                tstart = (iota + c * LANES) * BM
                te_vec = zi
                for e in range(E_ - 1):
                    eb = _lane_bcast(ends_vec, e, iota, bufi)
                    te_vec = te_vec + jnp.where(tstart >= eb, 1, 0)
                tile_v[pl.ds(c * LANES, LANES)] = te_vec
            pltpu.sync_copy(tile_v, te_hbm)

        carry0 = [zero] * E_
        carry1 = [zero] * E_
        for c in range(nch):
            v0 = ge0_v[pl.ds(base + c * LANES, LANES)]
            v1 = ge1_v[pl.ds(base + c * LANES, LANES)]
            pos0c = zi
            pos1c = zi
            for e in range(E_):
                m0 = jnp.where(v0 == fE[e], 1.0, 0.0)
                cs0 = _iprefix(m0, buf)
                r0 = cs0 - m0 + carry0[e]
                pos0c = pos0c + ((st0b[e] + r0.astype(jnp.int32))
                                 * m0.astype(jnp.int32))
                carry0[e] = carry0[e] + _lane_bcast(cs0, LANES - 1, iota, buf)
                m1 = jnp.where(v1 == fE[e], 1.0, 0.0)
                cs1 = _iprefix(m1, buf)
                r1 = cs1 - m1 + carry1[e]
                pos1c = pos1c + ((st1b[e] + r1.astype(jnp.int32))
                                 * m1.astype(jnp.int32))
                carry1[e] = carry1[e] + _lane_bcast(cs1, LANES - 1, iota, buf)
            p0_v[pl.ds(c * LANES, LANES)] = pos0c
            p1_v[pl.ds(c * LANES, LANES)] = pos1c

        pltpu.sync_copy(p0_v, pos0_hbm.at[pl.ds(base, tpw)])
        pltpu.sync_copy(p1_v, pos1_hbm.at[pl.ds(base, tpw)])

        cp.wait()
        c0 = pltpu.async_copy(xr_v, xp_hbm.at[p0_v], s0)
        c1 = pltpu.async_copy(xr_v, xp_hbm.at[p1_v], s0)
        w0cp = pltpu.async_copy(w0_v, wp_hbm.at[p0_v], s1)
        w1cp = pltpu.async_copy(w1_v, wp_hbm.at[p1_v], s1)
        c0.wait()
        c1.wait()
        w0cp.wait()
        w1cp.wait()

    return k(x, e0f, e1f, w0c, w1c)


def _moe_mm(tile_e, x_perm, w1b, w2b, w_col):
    def body(te_ref, x_ref, w1_ref, w2_ref, ws_ref, o_ref):
        h = jnp.dot(x_ref[...], w1_ref[0], preferred_element_type=jnp.float32)
        h = h * jax.nn.sigmoid(h)
        y = jnp.dot(h, w2_ref[0], preferred_element_type=jnp.float32)
        o_ref[...] = y * ws_ref[:, 0:1]

    grid_spec = pltpu.PrefetchScalarGridSpec(
        num_scalar_prefetch=1,
        grid=(NTILES,),
        in_specs=[
            pl.BlockSpec((BM, H_), lambda i, te: (i, 0)),
            pl.BlockSpec((1, H_, I_), lambda i, te: (te[i], 0, 0)),
            pl.BlockSpec((1, I_, H_), lambda i, te: (te[i], 0, 0)),
            pl.BlockSpec((BM, _EPAD), lambda i, te: (i, 0)),
        ],
        out_specs=pl.BlockSpec((BM, H_), lambda i, te: (i, 0)),
    )
    return pl.pallas_call(
        body,
        grid_spec=grid_spec,
        out_shape=jax.ShapeDtypeStruct((PPAD, H_), jnp.float32),
        compiler_params=pltpu.CompilerParams(
            dimension_semantics=("arbitrary",)),
    )(tile_e, x_perm, w1b, w2b, w_col)


def _sc_combine(y_perm, pos0, pos1):
    tpw = T_ // NW
    ch = 16
    nch = tpw // ch
    mesh = plsc.VectorSubcoreMesh(core_axis_name="c", subcore_axis_name="s",
                                  num_cores=NC, num_subcores=NS)

    @functools.partial(
        pl.kernel,
        out_type=jax.ShapeDtypeStruct((T_, H_), jnp.float32),
        mesh=mesh,
        scratch_types=[
            pltpu.VMEM((tpw,), jnp.int32),
            pltpu.VMEM((tpw,), jnp.int32),
            pltpu.VMEM((2, ch, H_), jnp.float32),
            pltpu.VMEM((2, ch, H_), jnp.float32),
            pltpu.SemaphoreType.DMA,
            pltpu.SemaphoreType.DMA,
        ],
    )
    def k(y_hbm, p0_hbm, p1_hbm, out_hbm, i0_v, i1_v, r0_v, r1_v, s0, s1):
        wid = lax.axis_index("s") * NC + lax.axis_index("c")
        base = wid * tpw
        pltpu.sync_copy(p0_hbm.at[pl.ds(base, tpw)], i0_v)
        pltpu.sync_copy(p1_hbm.at[pl.ds(base, tpw)], i1_v)

        def gathers(c):
            b = c % 2
            g0 = pltpu.async_copy(y_hbm.at[i0_v.at[pl.ds(c * ch, ch)]],
                                  r0_v.at[b], s0)
            g1 = pltpu.async_copy(y_hbm.at[i1_v.at[pl.ds(c * ch, ch)]],
                                  r1_v.at[b], s1)
            return g0, g1

        pend = gathers(0)
        for c in range(nch):
            pend[0].wait()
            pend[1].wait()
            if c + 1 < nch:
                pend = gathers(c + 1)
            b = c % 2

            def addrow(t, _):
                for kk in range(H_ // LANES):
                    sl = pl.ds(kk * LANES, LANES)
                    r0_v[b, t, sl] = r0_v[b, t, sl] + r1_v[b, t, sl]
                return 0

            lax.fori_loop(0, ch, addrow, 0)
            pltpu.sync_copy(r0_v.at[b],
                            out_hbm.at[pl.ds(base + c * ch, ch)])

    return k(y_perm, pos0, pos1)


def kernel(hidden_states, gate_w, experts_w1, experts_w2):
    router_logits = hidden_states @ gate_w
    probs = jax.nn.softmax(router_logits.astype(jnp.float32), axis=-1)
    i0f, i1f, w0, w1 = _router(probs)
    x_perm, w_perm, pos0, pos1, tile_e = _sc_dispatch(
        hidden_states, i0f.reshape(T_), i1f.reshape(T_), w0, w1)
    y_perm = _moe_mm(tile_e[:NTILES], x_perm, experts_w1, experts_w2, w_perm)
    return _sc_combine(y_perm, pos0, pos1)
